# Initial kernel scaffold; baseline (speedup 1.0000x reference)
import jax
import jax.numpy as jnp
from jax.experimental import pallas as pl


def kernel(scores, bbox_deltas, im_info):
    raise NotImplementedError("write your pallas kernel here")


# speedup vs baseline: 16.8951x; 16.8951x over previous
import numpy as np
import jax
import jax.numpy as jnp
from jax.experimental import pallas as pl
from jax.experimental.pallas import tpu as pltpu

_NUM_ANCHORS = 9
_PRE_NMS_TOP_N = 6000
_POST_NMS_TOP_N = 300
_NMS_THRESH = 0.7
_H, _W = 38, 63
_N = _H * _W * _NUM_ANCHORS
_ROWS = 176
_NP = _ROWS * 128
_NEG_INF = float("-inf")


def _anchor_table(base_size=16, ratios=(0.5, 1.0, 2.0), scales=(8, 16, 32)):
    def whctrs(a):
        w = a[2] - a[0] + 1.0
        h = a[3] - a[1] + 1.0
        return w, h, a[0] + 0.5 * (w - 1), a[1] + 0.5 * (h - 1)

    def mk(ws, hs, xc, yc):
        ws = np.asarray(ws, dtype=np.float64)[:, None]
        hs = np.asarray(hs, dtype=np.float64)[:, None]
        return np.hstack([xc - 0.5 * (ws - 1), yc - 0.5 * (hs - 1),
                          xc + 0.5 * (ws - 1), yc + 0.5 * (hs - 1)])

    base = np.array([0, 0, base_size - 1, base_size - 1], dtype=np.float64)
    w, h, xc, yc = whctrs(base)
    size = w * h
    sr = size / np.array(ratios, dtype=np.float64)
    ws = np.round(np.sqrt(sr))
    hs = np.round(ws * np.array(ratios, dtype=np.float64))
    ratio_anchors = mk(ws, hs, xc, yc)
    out = []
    for a in ratio_anchors:
        w, h, xc, yc = whctrs(a)
        out.append(mk(w * np.array(scales, dtype=np.float64),
                      h * np.array(scales, dtype=np.float64), xc, yc))
    return np.vstack(out).astype(np.float32)


_A9 = _anchor_table()


def _body(info_ref, sc_ref, dx_ref, dy_ref, dw_ref, dh_ref, out_ref,
          s_ref, x1_ref, y1_ref, x2_ref, y2_ref, ar_ref, idx_ref):
    f32 = jnp.float32
    im_h = info_ref[0]
    im_w = info_ref[1]
    scale = info_ref[2]

    ri = jax.lax.broadcasted_iota(jnp.int32, (_ROWS, 128), 0)
    ci = jax.lax.broadcasted_iota(jnp.int32, (_ROWS, 128), 1)
    n = ri * 128 + ci
    pos = n // _NUM_ANCHORS
    a = n - pos * _NUM_ANCHORS
    wi = pos - (pos // _W) * _W
    hi = pos // _W
    sx = wi.astype(f32) * 16.0
    sy = hi.astype(f32) * 16.0

    def a_sel(col):
        v = jnp.full((_ROWS, 128), float(_A9[0, col]), f32)
        for k in range(1, _NUM_ANCHORS):
            v = jnp.where(a == k, float(_A9[k, col]), v)
        return v

    ax1 = a_sel(0) + sx
    ay1 = a_sel(1) + sy
    ax2 = a_sel(2) + sx
    ay2 = a_sel(3) + sy

    widths = ax2 - ax1 + 1.0
    heights = ay2 - ay1 + 1.0
    ctr_x = ax1 + 0.5 * widths
    ctr_y = ay1 + 0.5 * heights

    dxv = dx_ref[...]
    dyv = dy_ref[...]
    dwv = jnp.clip(dw_ref[...], -10.0, 10.0)
    dhv = jnp.clip(dh_ref[...], -10.0, 10.0)

    pcx = dxv * widths + ctr_x
    pcy = dyv * heights + ctr_y
    pw = jnp.exp(dwv) * widths
    ph = jnp.exp(dhv) * heights

    x1 = jnp.clip(pcx - 0.5 * pw, 0.0, im_w - 1.0)
    y1 = jnp.clip(pcy - 0.5 * ph, 0.0, im_h - 1.0)
    x2 = jnp.clip(pcx + 0.5 * pw, 0.0, im_w - 1.0)
    y2 = jnp.clip(pcy + 0.5 * ph, 0.0, im_h - 1.0)

    ww = x2 - x1 + 1.0
    hh = y2 - y1 + 1.0
    areas = ww * hh
    min_sz = 16.0 * scale
    keep = (ww >= min_sz) & (hh >= min_sz)
    s = jnp.where(keep, sc_ref[...], _NEG_INF)

    key = jax.lax.bitcast_convert_type(s, jnp.int32)
    klo = jnp.int32(np.float32(_NEG_INF).view(np.int32))
    khi = jnp.int32(np.float32(1.0).view(np.int32))

    def bs_body(_, lohi):
        lo, hi = lohi
        mid = lo + (hi - lo + 1) // 2
        cnt = jnp.sum((key >= mid).astype(jnp.int32))
        feas = cnt >= _PRE_NMS_TOP_N
        return (jnp.where(feas, mid, lo), jnp.where(feas, hi, mid - 1))

    vkey, _ = jax.lax.fori_loop(0, 31, bs_body, (klo, khi))

    cnt_gt = jnp.sum((key > vkey).astype(f32))
    brem = jnp.float32(_PRE_NMS_TOP_N) - cnt_gt
    tie = (key == vkey).astype(f32)
    u_lane = (jax.lax.broadcasted_iota(jnp.int32, (128, 128), 0)
              < jax.lax.broadcasted_iota(jnp.int32, (128, 128), 1)).astype(f32)
    v_row = (jax.lax.broadcasted_iota(jnp.int32, (_ROWS, _ROWS), 1)
             < jax.lax.broadcasted_iota(jnp.int32, (_ROWS, _ROWS), 0)).astype(f32)
    p_lane = jnp.dot(tie, u_lane, preferred_element_type=f32)
    p_row = jnp.sum(jnp.dot(v_row, tie, preferred_element_type=f32),
                    axis=1, keepdims=True)
    prefix = p_lane + p_row
    elig = (key > vkey) | ((key == vkey) & (prefix < brem))

    s_ref[...] = jnp.where(elig, s, _NEG_INF)
    x1_ref[...] = x1
    y1_ref[...] = y1
    x2_ref[...] = x2
    y2_ref[...] = y2
    ar_ref[...] = areas
    idx_ref[...] = n.astype(f32)

    lane = jax.lax.broadcasted_iota(jnp.int32, (1, 128), 1)

    def nms_body(r, _):
        sv = s_ref[...]
        m = jnp.max(sv)
        valid = m > _NEG_INF
        nid = idx_ref[...]
        ii = jnp.min(jnp.where(sv == m, nid, jnp.float32(_NP)))
        sel = nid == ii

        x1v = x1_ref[...]
        y1v = y1_ref[...]
        x2v = x2_ref[...]
        y2v = y2_ref[...]
        arv = ar_ref[...]
        bx1 = jnp.sum(jnp.where(sel, x1v, 0.0))
        by1 = jnp.sum(jnp.where(sel, y1v, 0.0))
        bx2 = jnp.sum(jnp.where(sel, x2v, 0.0))
        by2 = jnp.sum(jnp.where(sel, y2v, 0.0))
        bar = jnp.sum(jnp.where(sel, arv, 0.0))

        xx1 = jnp.maximum(bx1, x1v)
        yy1 = jnp.maximum(by1, y1v)
        xx2 = jnp.minimum(bx2, x2v)
        yy2 = jnp.minimum(by2, y2v)
        w = jnp.maximum(0.0, xx2 - xx1 + 1.0)
        h = jnp.maximum(0.0, yy2 - yy1 + 1.0)
        inter = w * h
        iou = inter / (bar + arv - inter)
        supp = (iou > _NMS_THRESH) | sel
        s_ref[...] = jnp.where(supp, _NEG_INF, sv)

        vs = jnp.where(valid, 1.0, 0.0)
        row = vs * (jnp.where(lane == 1, bx1, 0.0)
                    + jnp.where(lane == 2, by1, 0.0)
                    + jnp.where(lane == 3, bx2, 0.0)
                    + jnp.where(lane == 4, by2, 0.0))
        out_ref[pl.ds(r, 1), :] = row
        return 0

    jax.lax.fori_loop(0, _POST_NMS_TOP_N, nms_body, 0)


def _run(info, sc, dx, dy, dw, dh):
    out = pl.pallas_call(
        _body,
        out_shape=jax.ShapeDtypeStruct((_POST_NMS_TOP_N + 4, 128), jnp.float32),
        in_specs=[
            pl.BlockSpec(memory_space=pltpu.SMEM),
            pl.BlockSpec(memory_space=pltpu.VMEM),
            pl.BlockSpec(memory_space=pltpu.VMEM),
            pl.BlockSpec(memory_space=pltpu.VMEM),
            pl.BlockSpec(memory_space=pltpu.VMEM),
            pl.BlockSpec(memory_space=pltpu.VMEM),
        ],
        out_specs=pl.BlockSpec(memory_space=pltpu.VMEM),
        scratch_shapes=[pltpu.VMEM((_ROWS, 128), jnp.float32)] * 7,
    )(info, sc, dx, dy, dw, dh)
    return out[:_POST_NMS_TOP_N, :5]


def kernel(scores, bbox_deltas, im_info):
    sc = jnp.transpose(scores[0, _NUM_ANCHORS:, :, :], (1, 2, 0)).reshape(-1)
    d = jnp.transpose(bbox_deltas[0], (1, 2, 0)).reshape(-1, 4)
    pad = _NP - _N
    sc = jnp.concatenate([sc, jnp.full((pad,), _NEG_INF, jnp.float32)])
    sc = sc.reshape(_ROWS, 128)
    zp = jnp.zeros((pad,), jnp.float32)
    dx = jnp.concatenate([d[:, 0], zp]).reshape(_ROWS, 128)
    dy = jnp.concatenate([d[:, 1], zp]).reshape(_ROWS, 128)
    dw = jnp.concatenate([d[:, 2], zp]).reshape(_ROWS, 128)
    dh = jnp.concatenate([d[:, 3], zp]).reshape(_ROWS, 128)
    return _run(im_info[0], sc, dx, dy, dw, dh)
